# baseline (speedup 1.0000x reference)
from functools import partial

import jax
import jax.numpy as jnp
from jax.experimental import pallas as pl
from jax.experimental.pallas import tpu as pltpu


def _gcn_body(x_ref, a_ref, w_ref, b_ref, o_ref, a16_s, xs_s, dinv_s,
              *, nb, T, V, Cin, Cout, S):
    i = pl.program_id(1)
    half = V // S

    @pl.when(i == 0)
    def _prep():
        for b in range(nb):
            a32 = a_ref[b]
            d = jnp.sum(a32, axis=-1, keepdims=True)
            dinv = jax.lax.rsqrt(d)
            dinv_s[b] = dinv
            a16_s[b] = a32.astype(jnp.bfloat16)
            xs_s[b] = (dinv * x_ref[b]).astype(jnp.bfloat16)

    w = w_ref[...]
    bias = b_ref[...]
    for b in range(nb):
        rows = pl.ds(i * half, half)
        a16h = a16_s[b, rows, :]
        dinvh = dinv_s[b, rows, :]
        y = jnp.dot(a16h, xs_s[b],
                    preferred_element_type=jnp.float32)
        for t in range(T):
            yt = y[:, t * Cin:(t + 1) * Cin].astype(jnp.bfloat16)
            proj = jnp.dot(yt, w,
                           preferred_element_type=jnp.float32)
            o_ref[b, t] = (dinvh * proj + bias).astype(o_ref.dtype)


def kernel(X, A, weight, bias):
    N, T, V, Cin = X.shape
    Cout = weight.shape[1]

    X2 = X.transpose(0, 2, 1, 3).reshape(N, V, T * Cin).astype(jnp.bfloat16)

    w16 = weight.astype(jnp.bfloat16)
    bias2 = bias.reshape(1, Cout)

    nb = next((c for c in (8, 4, 2, 1) if N % c == 0), 1)
    G = N // nb
    S = 2 if V % 2 == 0 else 1

    return pl.pallas_call(
        partial(_gcn_body, nb=nb, T=T, V=V, Cin=Cin, Cout=Cout, S=S),
        out_shape=jax.ShapeDtypeStruct((N, T, V, Cout), X.dtype),
        grid=(G, S),
        in_specs=[
            pl.BlockSpec((nb, V, T * Cin), lambda n, i: (n, 0, 0)),
            pl.BlockSpec((nb, V, V), lambda n, i: (n, 0, 0)),
            pl.BlockSpec((Cin, Cout), lambda n, i: (0, 0)),
            pl.BlockSpec((1, Cout), lambda n, i: (0, 0)),
        ],
        out_specs=pl.BlockSpec((nb, T, V // S, Cout),
                               lambda n, i: (n, 0, i, 0)),
        scratch_shapes=[
            pltpu.VMEM((nb, V, V), jnp.bfloat16),
            pltpu.VMEM((nb, V, T * Cin), jnp.bfloat16),
            pltpu.VMEM((nb, V, 1), jnp.float32),
        ],
        compiler_params=pltpu.CompilerParams(
            dimension_semantics=("parallel", "arbitrary")),
    )(X2, A, w16, bias2)


# speedup vs baseline: 1.2633x; 1.2633x over previous
from functools import partial

import jax
import jax.numpy as jnp
from jax.experimental import pallas as pl
from jax.experimental.pallas import tpu as pltpu


def _gcn_body(x_ref, a_ref, w_ref, b_ref, o_ref, *, nb, T, V, Cin, Cout):
    bias = b_ref[...]
    w = w_ref[...]

    for b in range(nb):
        a32 = a_ref[b]
        d = jnp.sum(a32, axis=-1, keepdims=True)
        dinv = jax.lax.rsqrt(d)
        a16 = a32.astype(jnp.bfloat16)

        xs = (dinv * x_ref[b]).astype(jnp.bfloat16)

        y = jnp.dot(a16, xs,
                    preferred_element_type=jnp.float32)

        for t in range(T):
            yt = y[:, t * Cin:(t + 1) * Cin].astype(jnp.bfloat16)
            proj = jnp.dot(yt, w,
                           preferred_element_type=jnp.float32)
            o_ref[b, t] = (dinv * proj + bias).astype(o_ref.dtype)


def kernel(X, A, weight, bias):
    N, T, V, Cin = X.shape
    Cout = weight.shape[1]

    X2 = X.transpose(0, 2, 1, 3).reshape(N, V, T * Cin).astype(jnp.bfloat16)

    w16 = weight.astype(jnp.bfloat16)
    bias2 = bias.reshape(1, Cout)

    nb = next((c for c in (8, 4, 2, 1) if N % c == 0), 1)
    G = N // nb

    return pl.pallas_call(
        partial(_gcn_body, nb=nb, T=T, V=V, Cin=Cin, Cout=Cout),
        out_shape=jax.ShapeDtypeStruct((N, T, V, Cout), X.dtype),
        grid=(G,),
        in_specs=[
            pl.BlockSpec((nb, V, T * Cin), lambda n: (n, 0, 0)),
            pl.BlockSpec((nb, V, V), lambda n: (n, 0, 0)),
            pl.BlockSpec((Cin, Cout), lambda n: (0, 0)),
            pl.BlockSpec((1, Cout), lambda n: (0, 0)),
        ],
        out_specs=pl.BlockSpec((nb, T, V, Cout), lambda n: (n, 0, 0, 0)),
        compiler_params=pltpu.CompilerParams(
            dimension_semantics=("parallel",)),
    )(X2, A, w16, bias2)


# speedup vs baseline: 1.3410x; 1.0615x over previous
from functools import partial

import jax
import jax.numpy as jnp
from jax.experimental import pallas as pl
from jax.experimental.pallas import tpu as pltpu


def _gcn_body(x_ref, a_ref, w_ref, b_ref, o_ref, *, nb, T, V, Cin, Cout):
    bias = b_ref[...]
    w = w_ref[...].astype(jnp.bfloat16)

    for b in range(nb):
        a32 = a_ref[b]
        d = jnp.sum(a32, axis=-1, keepdims=True)
        dinv = jax.lax.rsqrt(d)
        a16 = a32.astype(jnp.bfloat16)

        xs = (dinv * x_ref[b]).astype(jnp.bfloat16)

        y = jnp.dot(a16, xs,
                    preferred_element_type=jnp.float32)

        for t in range(T):
            yt = y[:, t * Cin:(t + 1) * Cin].astype(jnp.bfloat16)
            proj = jnp.dot(yt, w,
                           preferred_element_type=jnp.float32)
            o_ref[b, t] = (dinv * proj + bias).astype(o_ref.dtype)


def kernel(X, A, weight, bias):
    N, T, V, Cin = X.shape
    Cout = weight.shape[1]

    X2 = X.transpose(0, 2, 1, 3).reshape(N, V, T * Cin).astype(jnp.bfloat16)

    bias2 = bias.reshape(1, Cout)

    nb = next((c for c in (8, 4, 2, 1) if N % c == 0), 1)
    G = N // nb

    return pl.pallas_call(
        partial(_gcn_body, nb=nb, T=T, V=V, Cin=Cin, Cout=Cout),
        out_shape=jax.ShapeDtypeStruct((N, T, V, Cout), X.dtype),
        grid=(G,),
        in_specs=[
            pl.BlockSpec((nb, V, T * Cin), lambda n: (n, 0, 0)),
            pl.BlockSpec((nb, V, V), lambda n: (n, 0, 0)),
            pl.BlockSpec((Cin, Cout), lambda n: (0, 0)),
            pl.BlockSpec((1, Cout), lambda n: (0, 0)),
        ],
        out_specs=pl.BlockSpec((nb, T, V, Cout), lambda n: (n, 0, 0, 0)),
        compiler_params=pltpu.CompilerParams(
            dimension_semantics=("parallel",)),
    )(X2, A, weight, bias2)
